# baseline (device time: 31626 ns/iter reference)
import jax
import jax.numpy as jnp
from jax import lax
from jax.experimental import pallas as pl
from jax.experimental.pallas import tpu as pltpu

N_DEV = 8
N_LAYERS = 3
HALF = N_DEV // 2


def kernel(x, Win0, Wout0, Win1, Wout1, Win2, Wout2):
    b, d = x.shape
    chunk = b // N_DEV

    def body(x_ref, win0_ref, wout0_ref, win1_ref, wout1_ref, win2_ref,
             wout2_ref, out_ref, pbuf, rbuf, gbuf,
             red_send_sems, red_recv_sems, gat_send_sems, gat_recv_sems):
        my = lax.axis_index("i")

        barrier = pltpu.get_barrier_semaphore()
        for j in range(1, N_DEV):
            peer = lax.rem(my + j, N_DEV)
            pl.semaphore_signal(barrier, inc=1, device_id=(peer,),
                                device_id_type=pl.DeviceIdType.MESH)
        pl.semaphore_wait(barrier, N_DEV - 1)

        wins = (win0_ref, win1_ref, win2_ref)
        wouts = (wout0_ref, wout1_ref, wout2_ref)

        def wait_gather(l, src):
            r = pltpu.make_async_remote_copy(
                src_ref=gbuf.at[l, pl.ds(src * chunk, chunk), :],
                dst_ref=gbuf.at[l, pl.ds(src * chunk, chunk), :],
                send_sem=gat_send_sems.at[l],
                recv_sem=gat_recv_sems.at[l, src],
                device_id=(src,),
                device_id_type=pl.DeviceIdType.MESH,
            )
            r.wait_recv()

        for l in range(N_LAYERS):
            win = wins[l][...].astype(jnp.bfloat16)
            wout = wouts[l][...].astype(jnp.bfloat16)
            sends = []

            for half in range(2):
                lo = half * HALF
                if l == 0:
                    x_h = x_ref[pl.ds(lo * chunk, HALF * chunk), :].astype(
                        jnp.bfloat16)
                else:
                    for src in range(lo, lo + HALF):
                        @pl.when(my != src)
                        def _(src=src):
                            wait_gather(l - 1, src)
                    x_h = gbuf[l - 1, pl.ds(lo * chunk, HALF * chunk), :]

                h_h = jnp.dot(x_h, win, preferred_element_type=jnp.float32)
                h_h = jnp.maximum(h_h, 0.0).astype(jnp.bfloat16)
                p_h = jnp.dot(h_h, wout, preferred_element_type=jnp.float32)
                pbuf[l, pl.ds(lo * chunk, HALF * chunk), :] = p_h.astype(
                    jnp.bfloat16)

                for t in range(lo, lo + HALF):
                    @pl.when(my != t)
                    def _(t=t):
                        s = pltpu.make_async_remote_copy(
                            src_ref=pbuf.at[l, pl.ds(t * chunk, chunk), :],
                            dst_ref=rbuf.at[l, my],
                            send_sem=red_send_sems.at[l],
                            recv_sem=red_recv_sems.at[l, my],
                            device_id=(t,),
                            device_id_type=pl.DeviceIdType.MESH,
                        )
                        s.start()

            acc = pbuf[l, pl.ds(my * chunk, chunk), :].astype(jnp.float32)
            for j in range(1, N_DEV):
                src = lax.rem(my + N_DEV - j, N_DEV)
                r = pltpu.make_async_remote_copy(
                    src_ref=rbuf.at[l, src],
                    dst_ref=rbuf.at[l, src],
                    send_sem=red_send_sems.at[l],
                    recv_sem=red_recv_sems.at[l, src],
                    device_id=(src,),
                    device_id_type=pl.DeviceIdType.MESH,
                )
                r.wait_recv()
                acc = acc + rbuf[l, src].astype(jnp.float32)

            if l == N_LAYERS - 1:
                out_ref[...] = acc
            else:
                gbuf[l, pl.ds(my * chunk, chunk), :] = acc.astype(jnp.bfloat16)
                for j in range(1, N_DEV):
                    t = lax.rem(my + j, N_DEV)
                    s = pltpu.make_async_remote_copy(
                        src_ref=gbuf.at[l, pl.ds(my * chunk, chunk), :],
                        dst_ref=gbuf.at[l, pl.ds(my * chunk, chunk), :],
                        send_sem=gat_send_sems.at[l],
                        recv_sem=gat_recv_sems.at[l, my],
                        device_id=(t,),
                        device_id_type=pl.DeviceIdType.MESH,
                    )
                    s.start()
                    sends.append(s)

            for t in range(N_DEV):
                @pl.when(my != t)
                def _(t=t):
                    s = pltpu.make_async_remote_copy(
                        src_ref=pbuf.at[l, pl.ds(t * chunk, chunk), :],
                        dst_ref=rbuf.at[l, my],
                        send_sem=red_send_sems.at[l],
                        recv_sem=red_recv_sems.at[l, my],
                        device_id=(t,),
                        device_id_type=pl.DeviceIdType.MESH,
                    )
                    s.wait_send()
            for s in sends:
                s.wait_send()

    return pl.pallas_call(
        body,
        out_shape=jax.ShapeDtypeStruct((chunk, d), jnp.float32),
        in_specs=[pl.BlockSpec(memory_space=pltpu.VMEM)] * 7,
        out_specs=pl.BlockSpec(memory_space=pltpu.VMEM),
        scratch_shapes=[
            pltpu.VMEM((N_LAYERS, b, d), jnp.bfloat16),
            pltpu.VMEM((N_LAYERS, N_DEV, chunk, d), jnp.bfloat16),
            pltpu.VMEM((N_LAYERS, b, d), jnp.bfloat16),
            pltpu.SemaphoreType.DMA((N_LAYERS,)),
            pltpu.SemaphoreType.DMA((N_LAYERS, N_DEV)),
            pltpu.SemaphoreType.DMA((N_LAYERS,)),
            pltpu.SemaphoreType.DMA((N_LAYERS, N_DEV)),
        ],
        compiler_params=pltpu.CompilerParams(collective_id=0),
    )(x, Win0, Wout0, Win1, Wout1, Win2, Wout2)


# device time: 30992 ns/iter; 1.0205x vs baseline; 1.0205x over previous
import jax
import jax.numpy as jnp
from jax import lax
from jax.experimental import pallas as pl
from jax.experimental.pallas import tpu as pltpu

N_DEV = 8
N_LAYERS = 3
HALF = N_DEV // 2


def kernel(x, Win0, Wout0, Win1, Wout1, Win2, Wout2):
    b, d = x.shape
    chunk = b // N_DEV

    def body(x_ref, win0_ref, wout0_ref, win1_ref, wout1_ref, win2_ref,
             wout2_ref, out_ref, pbuf, rbuf, gbuf,
             red_send_sems, red_recv_sems, gat_send_sems, gat_recv_sems):
        my = lax.axis_index("i")

        barrier = pltpu.get_barrier_semaphore()
        for j in range(1, N_DEV):
            peer = lax.rem(my + j, N_DEV)
            pl.semaphore_signal(barrier, inc=1, device_id=(peer,),
                                device_id_type=pl.DeviceIdType.MESH)

        wins = (win0_ref, win1_ref, win2_ref)
        wouts = (wout0_ref, wout1_ref, wout2_ref)

        x0 = x_ref[...].astype(jnp.bfloat16)
        h0 = jnp.dot(x0, wins[0][...].astype(jnp.bfloat16),
                     preferred_element_type=jnp.float32)
        h0 = jnp.maximum(h0, 0.0).astype(jnp.bfloat16)
        p0 = jnp.dot(h0, wouts[0][...].astype(jnp.bfloat16),
                     preferred_element_type=jnp.float32)
        pbuf[0] = p0.astype(jnp.bfloat16)

        pl.semaphore_wait(barrier, N_DEV - 1)

        def wait_gather(l, src):
            r = pltpu.make_async_remote_copy(
                src_ref=gbuf.at[l, pl.ds(src * chunk, chunk), :],
                dst_ref=gbuf.at[l, pl.ds(src * chunk, chunk), :],
                send_sem=gat_send_sems.at[l],
                recv_sem=gat_recv_sems.at[l, src],
                device_id=(src,),
                device_id_type=pl.DeviceIdType.MESH,
            )
            r.wait_recv()

        for l in range(N_LAYERS):
            if l > 0:
                win = wins[l][...].astype(jnp.bfloat16)
                wout = wouts[l][...].astype(jnp.bfloat16)
            sends = []

            def send_chunk(t):
                @pl.when(my != t)
                def _():
                    s = pltpu.make_async_remote_copy(
                        src_ref=pbuf.at[l, pl.ds(t * chunk, chunk), :],
                        dst_ref=rbuf.at[l, my],
                        send_sem=red_send_sems.at[l],
                        recv_sem=red_recv_sems.at[l, my],
                        device_id=(t,),
                        device_id_type=pl.DeviceIdType.MESH,
                    )
                    s.start()

            if l == 0:
                for t in range(N_DEV):
                    send_chunk(t)
            else:
                GROUP = 2
                for g in range(N_DEV // GROUP):
                    lo = g * GROUP
                    for src in range(lo, lo + GROUP):
                        @pl.when(my != src)
                        def _(src=src):
                            wait_gather(l - 1, src)
                    x_h = gbuf[l - 1, pl.ds(lo * chunk, GROUP * chunk), :]
                    h_h = jnp.dot(x_h, win, preferred_element_type=jnp.float32)
                    h_h = jnp.maximum(h_h, 0.0).astype(jnp.bfloat16)
                    p_h = jnp.dot(h_h, wout, preferred_element_type=jnp.float32)
                    pbuf[l, pl.ds(lo * chunk, GROUP * chunk), :] = p_h.astype(
                        jnp.bfloat16)
                    for t in range(lo, lo + GROUP):
                        send_chunk(t)

            acc = pbuf[l, pl.ds(my * chunk, chunk), :].astype(jnp.float32)
            for j in range(1, N_DEV):
                src = lax.rem(my + N_DEV - j, N_DEV)
                r = pltpu.make_async_remote_copy(
                    src_ref=rbuf.at[l, src],
                    dst_ref=rbuf.at[l, src],
                    send_sem=red_send_sems.at[l],
                    recv_sem=red_recv_sems.at[l, src],
                    device_id=(src,),
                    device_id_type=pl.DeviceIdType.MESH,
                )
                r.wait_recv()
                acc = acc + rbuf[l, src].astype(jnp.float32)

            if l == N_LAYERS - 1:
                out_ref[...] = acc
            else:
                gbuf[l, pl.ds(my * chunk, chunk), :] = acc.astype(jnp.bfloat16)
                for j in range(1, N_DEV):
                    t = lax.rem(my + j, N_DEV)
                    s = pltpu.make_async_remote_copy(
                        src_ref=gbuf.at[l, pl.ds(my * chunk, chunk), :],
                        dst_ref=gbuf.at[l, pl.ds(my * chunk, chunk), :],
                        send_sem=gat_send_sems.at[l],
                        recv_sem=gat_recv_sems.at[l, my],
                        device_id=(t,),
                        device_id_type=pl.DeviceIdType.MESH,
                    )
                    s.start()
                    sends.append(s)

            for t in range(N_DEV):
                @pl.when(my != t)
                def _(t=t):
                    s = pltpu.make_async_remote_copy(
                        src_ref=pbuf.at[l, pl.ds(t * chunk, chunk), :],
                        dst_ref=rbuf.at[l, my],
                        send_sem=red_send_sems.at[l],
                        recv_sem=red_recv_sems.at[l, my],
                        device_id=(t,),
                        device_id_type=pl.DeviceIdType.MESH,
                    )
                    s.wait_send()
            for s in sends:
                s.wait_send()

    return pl.pallas_call(
        body,
        out_shape=jax.ShapeDtypeStruct((chunk, d), jnp.float32),
        in_specs=[pl.BlockSpec(memory_space=pltpu.VMEM)] * 7,
        out_specs=pl.BlockSpec(memory_space=pltpu.VMEM),
        scratch_shapes=[
            pltpu.VMEM((N_LAYERS, b, d), jnp.bfloat16),
            pltpu.VMEM((N_LAYERS, N_DEV, chunk, d), jnp.bfloat16),
            pltpu.VMEM((N_LAYERS, b, d), jnp.bfloat16),
            pltpu.SemaphoreType.DMA((N_LAYERS,)),
            pltpu.SemaphoreType.DMA((N_LAYERS, N_DEV)),
            pltpu.SemaphoreType.DMA((N_LAYERS,)),
            pltpu.SemaphoreType.DMA((N_LAYERS, N_DEV)),
        ],
        compiler_params=pltpu.CompilerParams(collective_id=0),
    )(x, Win0, Wout0, Win1, Wout1, Win2, Wout2)


# device time: 30939 ns/iter; 1.0222x vs baseline; 1.0017x over previous
import jax
import jax.numpy as jnp
from jax import lax
from jax.experimental import pallas as pl
from jax.experimental.pallas import tpu as pltpu

N_DEV = 8
N_LAYERS = 3
HALF = N_DEV // 2


def kernel(x, Win0, Wout0, Win1, Wout1, Win2, Wout2):
    b, d = x.shape
    chunk = b // N_DEV

    def body(x_ref, win0_ref, wout0_ref, win1_ref, wout1_ref, win2_ref,
             wout2_ref, out_ref, pbuf, rbuf, gbuf,
             red_send_sems, red_recv_sems, gat_send_sems, gat_recv_sems):
        my = lax.axis_index("i")

        barrier = pltpu.get_barrier_semaphore()
        for j in range(1, N_DEV):
            peer = lax.rem(my + j, N_DEV)
            pl.semaphore_signal(barrier, inc=1, device_id=(peer,),
                                device_id_type=pl.DeviceIdType.MESH)

        wins = (win0_ref, win1_ref, win2_ref)
        wouts = (wout0_ref, wout1_ref, wout2_ref)

        x0 = x_ref[...].astype(jnp.bfloat16)
        h0 = jnp.dot(x0, wins[0][...].astype(jnp.bfloat16),
                     preferred_element_type=jnp.float32)
        h0 = jnp.maximum(h0, 0.0).astype(jnp.bfloat16)
        p0 = jnp.dot(h0, wouts[0][...].astype(jnp.bfloat16),
                     preferred_element_type=jnp.float32)
        pbuf[0] = p0.astype(jnp.bfloat16)

        pl.semaphore_wait(barrier, N_DEV - 1)

        def wait_gather(l, src):
            r = pltpu.make_async_remote_copy(
                src_ref=gbuf.at[l, pl.ds(src * chunk, chunk), :],
                dst_ref=gbuf.at[l, pl.ds(src * chunk, chunk), :],
                send_sem=gat_send_sems.at[l],
                recv_sem=gat_recv_sems.at[l, src],
                device_id=(src,),
                device_id_type=pl.DeviceIdType.MESH,
            )
            r.wait_recv()

        for l in range(N_LAYERS):
            if l > 0:
                win = wins[l][...].astype(jnp.bfloat16)
                wout = wouts[l][...].astype(jnp.bfloat16)
            sends = []

            def send_chunk(t):
                @pl.when(my != t)
                def _():
                    s = pltpu.make_async_remote_copy(
                        src_ref=pbuf.at[l, pl.ds(t * chunk, chunk), :],
                        dst_ref=rbuf.at[l, my],
                        send_sem=red_send_sems.at[l],
                        recv_sem=red_recv_sems.at[l, my],
                        device_id=(t,),
                        device_id_type=pl.DeviceIdType.MESH,
                    )
                    s.start()

            if l == 0:
                for t in range(N_DEV):
                    send_chunk(t)
            else:
                GROUP = 1
                for g in range(N_DEV // GROUP):
                    lo = g * GROUP
                    for src in range(lo, lo + GROUP):
                        @pl.when(my != src)
                        def _(src=src):
                            wait_gather(l - 1, src)
                    x_h = gbuf[l - 1, pl.ds(lo * chunk, GROUP * chunk), :]
                    h_h = jnp.dot(x_h, win, preferred_element_type=jnp.float32)
                    h_h = jnp.maximum(h_h, 0.0).astype(jnp.bfloat16)
                    p_h = jnp.dot(h_h, wout, preferred_element_type=jnp.float32)
                    pbuf[l, pl.ds(lo * chunk, GROUP * chunk), :] = p_h.astype(
                        jnp.bfloat16)
                    for t in range(lo, lo + GROUP):
                        send_chunk(t)

            acc = pbuf[l, pl.ds(my * chunk, chunk), :].astype(jnp.float32)
            for j in range(1, N_DEV):
                src = lax.rem(my + N_DEV - j, N_DEV)
                r = pltpu.make_async_remote_copy(
                    src_ref=rbuf.at[l, src],
                    dst_ref=rbuf.at[l, src],
                    send_sem=red_send_sems.at[l],
                    recv_sem=red_recv_sems.at[l, src],
                    device_id=(src,),
                    device_id_type=pl.DeviceIdType.MESH,
                )
                r.wait_recv()
                acc = acc + rbuf[l, src].astype(jnp.float32)

            if l == N_LAYERS - 1:
                out_ref[...] = acc
            else:
                gbuf[l, pl.ds(my * chunk, chunk), :] = acc.astype(jnp.bfloat16)
                for j in range(1, N_DEV):
                    t = lax.rem(my + j, N_DEV)
                    s = pltpu.make_async_remote_copy(
                        src_ref=gbuf.at[l, pl.ds(my * chunk, chunk), :],
                        dst_ref=gbuf.at[l, pl.ds(my * chunk, chunk), :],
                        send_sem=gat_send_sems.at[l],
                        recv_sem=gat_recv_sems.at[l, my],
                        device_id=(t,),
                        device_id_type=pl.DeviceIdType.MESH,
                    )
                    s.start()
                    sends.append(s)

            for t in range(N_DEV):
                @pl.when(my != t)
                def _(t=t):
                    s = pltpu.make_async_remote_copy(
                        src_ref=pbuf.at[l, pl.ds(t * chunk, chunk), :],
                        dst_ref=rbuf.at[l, my],
                        send_sem=red_send_sems.at[l],
                        recv_sem=red_recv_sems.at[l, my],
                        device_id=(t,),
                        device_id_type=pl.DeviceIdType.MESH,
                    )
                    s.wait_send()
            for s in sends:
                s.wait_send()

    return pl.pallas_call(
        body,
        out_shape=jax.ShapeDtypeStruct((chunk, d), jnp.float32),
        in_specs=[pl.BlockSpec(memory_space=pltpu.VMEM)] * 7,
        out_specs=pl.BlockSpec(memory_space=pltpu.VMEM),
        scratch_shapes=[
            pltpu.VMEM((N_LAYERS, b, d), jnp.bfloat16),
            pltpu.VMEM((N_LAYERS, N_DEV, chunk, d), jnp.bfloat16),
            pltpu.VMEM((N_LAYERS, b, d), jnp.bfloat16),
            pltpu.SemaphoreType.DMA((N_LAYERS,)),
            pltpu.SemaphoreType.DMA((N_LAYERS, N_DEV)),
            pltpu.SemaphoreType.DMA((N_LAYERS,)),
            pltpu.SemaphoreType.DMA((N_LAYERS, N_DEV)),
        ],
        compiler_params=pltpu.CompilerParams(collective_id=0),
    )(x, Win0, Wout0, Win1, Wout1, Win2, Wout2)
